# Initial kernel scaffold; baseline (speedup 1.0000x reference)
import jax
import jax.numpy as jnp
from jax.experimental import pallas as pl


def kernel(inputs, adj, w1, w2):
    raise NotImplementedError("write your pallas kernel here")


# speedup vs baseline: 32.4803x; 32.4803x over previous
import jax
import jax.numpy as jnp
from jax.experimental import pallas as pl
from jax.experimental.pallas import tpu as pltpu

N = 512
D = 128
H = 64


def _prep_body(z_ref, adj_ref, w1_ref, w_ref, at_ref, bt_ref):
    adj = adj_ref[:]
    row = jax.lax.broadcasted_iota(jnp.int32, (N, N), 0)
    col = jax.lax.broadcasted_iota(jnp.int32, (N, N), 1)
    tri = (col < row).astype(jnp.float32)
    s = jnp.dot(tri, adj, preferred_element_type=jnp.float32)
    w_ref[:] = tri * jax.lax.rsqrt(jnp.maximum(s, 1.0))
    at_ref[:] = adj.T
    b = jnp.dot(z_ref[:], w1_ref[:D, :], preferred_element_type=jnp.float32)
    bt_ref[:] = b.T.reshape(H, 1, N)


def _acc_body(w2_ref, w_ref, at_ref, bt_ref, r_ref):
    h = pl.program_id(0)

    @pl.when(h == 0)
    def _():
        r_ref[:] = jnp.zeros((N, N), jnp.float32)

    w = w_ref[:]
    m = w * bt_ref[0, 0, :][None, :]
    y = jnp.dot(m, at_ref[:], preferred_element_type=jnp.float32)
    r_ref[:] += w2_ref[h] * jnp.maximum(w * y, 0.0)


def _final_body(z_ref, w_ref, at_ref, r_ref, out_ref):
    z = z_ref[:]
    w = w_ref[:]
    p = jnp.dot(w * r_ref[:], at_ref[:], preferred_element_type=jnp.float32)
    sup = w * p
    x = jnp.dot(z, z.T, preferred_element_type=jnp.float32)
    out_ref[:] = x + 0.5 * (sup + sup.T)


def kernel(inputs, adj, w1, w2):
    f32 = jnp.float32
    w_mat, at, bt = pl.pallas_call(
        _prep_body,
        out_shape=(
            jax.ShapeDtypeStruct((N, N), f32),
            jax.ShapeDtypeStruct((N, N), f32),
            jax.ShapeDtypeStruct((H, 1, N), f32),
        ),
    )(inputs, adj, w1)

    r = pl.pallas_call(
        _acc_body,
        grid=(H,),
        in_specs=[
            pl.BlockSpec(memory_space=pltpu.SMEM),
            pl.BlockSpec((N, N), lambda h: (0, 0)),
            pl.BlockSpec((N, N), lambda h: (0, 0)),
            pl.BlockSpec((1, 1, N), lambda h: (h, 0, 0)),
        ],
        out_specs=pl.BlockSpec((N, N), lambda h: (0, 0)),
        out_shape=jax.ShapeDtypeStruct((N, N), f32),
    )(w2.reshape(H), w_mat, at, bt)

    out = pl.pallas_call(
        _final_body,
        out_shape=jax.ShapeDtypeStruct((N, N), f32),
    )(inputs, w_mat, at, r)
    return out


# speedup vs baseline: 35.4731x; 1.0921x over previous
import jax
import jax.numpy as jnp
from jax.experimental import pallas as pl
from jax.experimental.pallas import tpu as pltpu

N = 512
D = 128
H = 64


def _prep_body(z_ref, adj_ref, w1_ref, w_ref, at_ref, bt_ref):
    adj = adj_ref[:]
    row = jax.lax.broadcasted_iota(jnp.int32, (N, N), 0)
    col = jax.lax.broadcasted_iota(jnp.int32, (N, N), 1)
    tri = (col < row).astype(jnp.float32)
    s = jnp.dot(tri, adj, preferred_element_type=jnp.float32)
    w_ref[:] = tri * jax.lax.rsqrt(jnp.maximum(s, 1.0))
    at_ref[:] = adj.T
    b = jnp.dot(z_ref[:], w1_ref[:D, :], preferred_element_type=jnp.float32)
    bt_ref[:] = b.T.reshape(H, 1, N)


_BS = 128


def _acc_body(w2_ref, w_ref, at_ref, bt_ref, r_ref):
    h = pl.program_id(0)

    @pl.when(h == 0)
    def _():
        r_ref[:] = jnp.zeros((N, N), jnp.float32)

    b = bt_ref[0, 0, :]
    s = w2_ref[h]
    for blk in range(N // _BS):
        r0 = blk * _BS
        ext = r0 + _BS
        wb = w_ref[r0:ext, :ext]
        mb = wb * b[:ext][None, :]
        yb = jnp.dot(mb, at_ref[:ext, :ext], preferred_element_type=jnp.float32)
        r_ref[r0:ext, :ext] += s * jnp.maximum(wb * yb, 0.0)


def _final_body(z_ref, w_ref, at_ref, r_ref, out_ref):
    z = z_ref[:]
    w = w_ref[:]
    p = jnp.dot(w * r_ref[:], at_ref[:], preferred_element_type=jnp.float32)
    sup = w * p
    x = jnp.dot(z, z.T, preferred_element_type=jnp.float32)
    out_ref[:] = x + 0.5 * (sup + sup.T)


def kernel(inputs, adj, w1, w2):
    f32 = jnp.float32
    w_mat, at, bt = pl.pallas_call(
        _prep_body,
        out_shape=(
            jax.ShapeDtypeStruct((N, N), f32),
            jax.ShapeDtypeStruct((N, N), f32),
            jax.ShapeDtypeStruct((H, 1, N), f32),
        ),
    )(inputs, adj, w1)

    r = pl.pallas_call(
        _acc_body,
        grid=(H,),
        in_specs=[
            pl.BlockSpec(memory_space=pltpu.SMEM),
            pl.BlockSpec((N, N), lambda h: (0, 0)),
            pl.BlockSpec((N, N), lambda h: (0, 0)),
            pl.BlockSpec((1, 1, N), lambda h: (h, 0, 0)),
        ],
        out_specs=pl.BlockSpec((N, N), lambda h: (0, 0)),
        out_shape=jax.ShapeDtypeStruct((N, N), f32),
    )(w2.reshape(H), w_mat, at, bt)

    out = pl.pallas_call(
        _final_body,
        out_shape=jax.ShapeDtypeStruct((N, N), f32),
    )(inputs, w_mat, at, r)
    return out
